# baseline (speedup 1.0000x reference)
import jax
import jax.numpy as jnp
from jax import lax
from jax.experimental import pallas as pl
from jax.experimental.pallas import tpu as pltpu
from jax.experimental.pallas import tpu_sc as plsc

NTOKEN = 256
MAX_PREFIX = 7
D_MODEL = 2048
SEQ_LEN = 2048
BATCH = 4

ROWS = SEQ_LEN * BATCH
NC, NS, L = 2, 16, 16
NW = NC * NS
ROWS_PER_W = ROWS // NW
GROUPS = ROWS_PER_W // L
PAD = 32
NSEG = MAX_PREFIX + 1
BUF = L * D_MODEL
NBUF = 3


def _body(src_hbm, out_hbm, src_v, buf0, buf1, buf2, sem0, sem1, sem2):
    wid = lax.axis_index("s") * NC + lax.axis_index("c")
    zeros16_i = jnp.zeros((L,), jnp.int32)
    zeros16_f = jnp.zeros((L,), jnp.float32)
    ones16_f = jnp.ones((L,), jnp.float32)
    lane = lax.iota(jnp.int32, L)
    lane_off = lane * D_MODEL

    src_v[pl.ds(0, L)] = zeros16_i
    src_v[pl.ds(L, L)] = zeros16_i
    pltpu.sync_copy(src_hbm, src_v.at[pl.ds(PAD, ROWS)])

    bufs = (buf0, buf1, buf2)
    sems = (sem0, sem1, sem2)

    @pl.loop(0, BUF // L)
    def _zero(c):
        for bf in bufs:
            bf[pl.ds(c * L, L)] = zeros16_f

    row_base = wid * ROWS_PER_W

    def seg_cols_vals(r0):
        out = []
        for i in range(NSEG):
            cols = src_v[pl.ds(PAD + r0 - 4 * i, L)] + (i * NTOKEN)
            vals = jnp.where(r0 + lane >= 4 * i, ones16_f, zeros16_f)
            out.append((lane_off + cols, vals))
        return out

    copies = [None] * NBUF
    for k in range(GROUPS):
        b = k % NBUF
        buf = bufs[b]
        r0 = row_base + k * L
        if k >= NBUF:
            copies[b].wait()
            r0_old = row_base + (k - NBUF) * L
            for idx, _ in seg_cols_vals(r0_old):
                plsc.store_scatter(buf, [idx], zeros16_f)
        for idx, vals in seg_cols_vals(r0):
            plsc.store_scatter(buf, [idx], vals)
        copies[b] = pltpu.async_copy(
            buf, out_hbm.at[pl.ds(r0 * D_MODEL, BUF)], sems[b]
        )
    for b in range(NBUF):
        copies[(GROUPS - NBUF + b) % NBUF].wait()


@jax.jit
def kernel(src):
    mesh = plsc.VectorSubcoreMesh(
        core_axis_name="c", subcore_axis_name="s", num_cores=NC, num_subcores=NS
    )
    k = pl.kernel(
        _body,
        out_type=jax.ShapeDtypeStruct((ROWS * D_MODEL,), jnp.float32),
        mesh=mesh,
        scratch_types=[
            pltpu.VMEM((PAD + ROWS,), jnp.int32),
            *[pltpu.VMEM((BUF,), jnp.float32) for _ in range(NBUF)],
            *[pltpu.SemaphoreType.DMA for _ in range(NBUF)],
        ],
        compiler_params=pltpu.CompilerParams(needs_layout_passes=False),
    )
    out = k(src.reshape(ROWS))
    return out.reshape(SEQ_LEN, BATCH, D_MODEL)


# speedup vs baseline: 2.6647x; 2.6647x over previous
import jax
import jax.numpy as jnp
from jax import lax
from jax.experimental import pallas as pl
from jax.experimental.pallas import tpu as pltpu
from jax.experimental.pallas import tpu_sc as plsc

NTOKEN = 256
MAX_PREFIX = 7
D_MODEL = 2048
SEQ_LEN = 2048
BATCH = 4

ROWS = SEQ_LEN * BATCH
NC, NS, L = 2, 16, 16
NW = NC * NS
T_PER_W = SEQ_LEN // NW
T_G = L // BATCH
GROUPS = T_PER_W // T_G
PAD = 32
NSEG = MAX_PREFIX + 1
NBUF = 3


def _body(src_hbm, out_hbm, src_v, buf0, buf1, buf2, sem0, sem1, sem2):
    wid = lax.axis_index("s") * NC + lax.axis_index("c")
    zeros16_i = jnp.zeros((L,), jnp.int32)
    zeros16_f = jnp.zeros((L,), jnp.float32)
    ones16_f = jnp.ones((L,), jnp.float32)
    lane = lax.iota(jnp.int32, L)
    t_loc = lane >> 2
    b_loc = lane & 3

    src_v[pl.ds(0, L)] = zeros16_i
    src_v[pl.ds(L, L)] = zeros16_i
    pltpu.sync_copy(src_hbm, src_v.at[pl.ds(PAD, ROWS)])

    bufs = (buf0, buf1, buf2)
    sems = (sem0, sem1, sem2)

    @pl.loop(0, D_MODEL // L)
    def _zero(c):
        for bf in bufs:
            for tt in range(T_G):
                for bb in range(BATCH):
                    bf[tt, bb, pl.ds(c * L, L)] = zeros16_f

    t_base = wid * T_PER_W

    def seg_cols_vals(r0):
        out = []
        for i in range(NSEG):
            cols = src_v[pl.ds(PAD + r0 - BATCH * i, L)] + (i * NTOKEN)
            vals = jnp.where(r0 + lane >= BATCH * i, ones16_f, zeros16_f)
            out.append((cols, vals))
        return out

    copies = [None] * NBUF
    for k in range(GROUPS):
        b = k % NBUF
        buf = bufs[b]
        t0 = t_base + k * T_G
        r0 = t0 * BATCH
        if k >= NBUF:
            copies[b].wait()
            r0_old = r0 - NBUF * L
            for cols, _ in seg_cols_vals(r0_old):
                plsc.store_scatter(buf, [t_loc, b_loc, cols], zeros16_f)
        for cols, vals in seg_cols_vals(r0):
            plsc.store_scatter(buf, [t_loc, b_loc, cols], vals)
        copies[b] = pltpu.async_copy(buf, out_hbm.at[pl.ds(t0, T_G)], sems[b])
    for b in range(NBUF):
        copies[(GROUPS - NBUF + b) % NBUF].wait()


@jax.jit
def kernel(src):
    mesh = plsc.VectorSubcoreMesh(
        core_axis_name="c", subcore_axis_name="s", num_cores=NC, num_subcores=NS
    )
    k = pl.kernel(
        _body,
        out_type=jax.ShapeDtypeStruct((SEQ_LEN, BATCH, D_MODEL), jnp.float32),
        mesh=mesh,
        scratch_types=[
            pltpu.VMEM((PAD + ROWS,), jnp.int32),
            *[pltpu.VMEM((T_G, BATCH, D_MODEL), jnp.float32) for _ in range(NBUF)],
            *[pltpu.SemaphoreType.DMA for _ in range(NBUF)],
        ],
        compiler_params=pltpu.CompilerParams(needs_layout_passes=False),
    )
    return k(src.reshape(ROWS))


# speedup vs baseline: 2.9268x; 1.0983x over previous
import jax
import jax.numpy as jnp
from jax import lax
from jax.experimental import pallas as pl
from jax.experimental.pallas import tpu as pltpu
from jax.experimental.pallas import tpu_sc as plsc

NTOKEN = 256
MAX_PREFIX = 7
D_MODEL = 2048
SEQ_LEN = 2048
BATCH = 4

NC, NS, L = 2, 16, 16
NW = NC * NS
T_PER_W = SEQ_LEN // NW
T_G = L // BATCH
GROUPS = T_PER_W // T_G
NSEG = MAX_PREFIX + 1
NBUF = 3
T_WIN = T_PER_W + 8


def _body(src_hbm, out_hbm, src_v, buf0, buf1, buf2, src_sem, sem0, sem1, sem2):
    wid = lax.axis_index("s") * NC + lax.axis_index("c")
    zeros16_f = jnp.zeros((L,), jnp.float32)
    ones16_f = jnp.ones((L,), jnp.float32)
    zero16_i = jnp.zeros((L,), jnp.int32)
    lane = lax.iota(jnp.int32, L)
    t_loc = lane >> 2
    b_loc = lane & 3

    t_base = wid * T_PER_W
    t_lo = pl.multiple_of(jnp.maximum(t_base - 8, 0), 8)
    src_cp = pltpu.async_copy(src_hbm.at[pl.ds(t_lo, T_WIN)], src_v, src_sem)

    bufs = (buf0, buf1, buf2)
    sems = (sem0, sem1, sem2)

    def zero_buf(bf):
        @pl.loop(0, D_MODEL // L)
        def _zero(c):
            for tt in range(T_G):
                for bb in range(BATCH):
                    bf[tt, bb, pl.ds(c * L, L)] = zeros16_f

    zero_buf(buf0)
    src_cp.wait()

    def seg_cols_vals(t0):
        out = []
        for i in range(NSEG):
            t_idx = jnp.maximum(t0 - i + t_loc - t_lo, zero16_i)
            cols = plsc.load_gather(src_v, [t_idx, b_loc]) + (i * NTOKEN)
            vals = jnp.where(t0 + t_loc >= i, ones16_f, zeros16_f)
            out.append((cols, vals))
        return out

    copies = [None] * NBUF
    for k in range(GROUPS):
        b = k % NBUF
        buf = bufs[b]
        t0 = t_base + k * T_G
        if k >= NBUF:
            copies[b].wait()
            for cols, _ in seg_cols_vals(t0 - NBUF * T_G):
                plsc.store_scatter(buf, [t_loc, b_loc, cols], zeros16_f)
        for cols, vals in seg_cols_vals(t0):
            plsc.store_scatter(buf, [t_loc, b_loc, cols], vals)
        copies[b] = pltpu.async_copy(buf, out_hbm.at[pl.ds(t0, T_G)], sems[b])
        if k + 1 < NBUF:
            zero_buf(bufs[k + 1])
    for b in range(NBUF):
        copies[(GROUPS - NBUF + b) % NBUF].wait()


@jax.jit
def kernel(src):
    mesh = plsc.VectorSubcoreMesh(
        core_axis_name="c", subcore_axis_name="s", num_cores=NC, num_subcores=NS
    )
    k = pl.kernel(
        _body,
        out_type=jax.ShapeDtypeStruct((SEQ_LEN, BATCH, D_MODEL), jnp.float32),
        mesh=mesh,
        scratch_types=[
            pltpu.VMEM((T_WIN, BATCH), jnp.int32),
            *[pltpu.VMEM((T_G, BATCH, D_MODEL), jnp.float32) for _ in range(NBUF)],
            pltpu.SemaphoreType.DMA,
            *[pltpu.SemaphoreType.DMA for _ in range(NBUF)],
        ],
        compiler_params=pltpu.CompilerParams(needs_layout_passes=False),
    )
    return k(src)


# speedup vs baseline: 2.9287x; 1.0006x over previous
import jax
import jax.numpy as jnp
from jax import lax
from jax.experimental import pallas as pl
from jax.experimental.pallas import tpu as pltpu
from jax.experimental.pallas import tpu_sc as plsc

NTOKEN = 256
MAX_PREFIX = 7
D_MODEL = 2048
SEQ_LEN = 2048
BATCH = 4

NC, NS, L = 2, 16, 16
NW = NC * NS
T_PER_W = SEQ_LEN // NW
T_G = L // BATCH
GROUPS = T_PER_W // T_G
NSEG = MAX_PREFIX + 1
NBUF = 3
T_WIN = T_PER_W + 8


def _body(src_hbm, out_hbm, src_v, buf0, buf1, buf2, src_sem, sem0, sem1, sem2):
    wid = lax.axis_index("s") * NC + lax.axis_index("c")
    zeros16_f = jnp.zeros((L,), jnp.float32)
    ones16_f = jnp.ones((L,), jnp.float32)
    zero16_i = jnp.zeros((L,), jnp.int32)
    lane = lax.iota(jnp.int32, L)
    t_loc = lane >> 2
    b_loc = lane & 3

    t_base = wid * T_PER_W
    t_lo = pl.multiple_of(jnp.maximum(t_base - 8, 0), 8)
    src_cp = pltpu.async_copy(src_hbm.at[pl.ds(t_lo, T_WIN)], src_v, src_sem)

    bufs = (buf0, buf1, buf2)
    sems = (sem0, sem1, sem2)

    def zero_buf(bf):
        @pl.loop(0, D_MODEL // L)
        def _zero(c):
            for tt in range(T_G):
                for bb in range(BATCH):
                    bf[tt, bb, pl.ds(c * L, L)] = zeros16_f

    zero_buf(buf0)
    src_cp.wait()

    def seg_cols_vals(t0):
        out = []
        for i in range(NSEG):
            t_idx = jnp.maximum(t0 - i + t_loc - t_lo, zero16_i)
            cols = plsc.load_gather(src_v, [t_idx, b_loc]) + (i * NTOKEN)
            vals = jnp.where(t0 + t_loc >= i, ones16_f, zeros16_f)
            out.append((cols, vals))
        return out

    copies = [None] * NBUF
    for k in range(GROUPS):
        b = k % NBUF
        buf = bufs[b]
        t0 = t_base + k * T_G
        if k >= NBUF:
            copies[b].wait()
            for cols, _ in seg_cols_vals(t0 - NBUF * T_G):
                plsc.store_scatter(buf, [t_loc, b_loc, cols], zeros16_f)
        for cols, vals in seg_cols_vals(t0):
            plsc.store_scatter(buf, [t_loc, b_loc, cols], vals)
        copies[b] = pltpu.async_copy(buf, out_hbm.at[pl.ds(t0, T_G)], sems[b])
        if k + 1 < NBUF:
            zero_buf(bufs[k + 1])
    for b in range(NBUF):
        copies[(GROUPS - NBUF + b) % NBUF].wait()


@jax.jit
def kernel(src):
    mesh = plsc.VectorSubcoreMesh(
        core_axis_name="c", subcore_axis_name="s", num_cores=NC, num_subcores=NS
    )
    k = pl.kernel(
        _body,
        out_type=jax.ShapeDtypeStruct((SEQ_LEN, BATCH, D_MODEL), jnp.float32),
        mesh=mesh,
        scratch_types=[
            pltpu.VMEM((T_WIN, BATCH), jnp.int32),
            *[pltpu.VMEM((T_G, BATCH, D_MODEL), jnp.float32) for _ in range(NBUF)],
            pltpu.SemaphoreType.DMA,
            *[pltpu.SemaphoreType.DMA for _ in range(NBUF)],
        ],
        compiler_params=pltpu.CompilerParams(
            needs_layout_passes=False, use_tc_tiling_on_sc=True
        ),
    )
    return k(src)


# speedup vs baseline: 3.0319x; 1.0353x over previous
import jax
import jax.numpy as jnp
from jax import lax
from jax.experimental import pallas as pl
from jax.experimental.pallas import tpu as pltpu
from jax.experimental.pallas import tpu_sc as plsc

NTOKEN = 256
MAX_PREFIX = 7
D_MODEL = 2048
SEQ_LEN = 2048
BATCH = 4

NC, NS, L = 2, 16, 16
NW = NC * NS
T_PER_W = SEQ_LEN // NW
T_G = L // BATCH
GROUPS = T_PER_W // T_G
NSEG = MAX_PREFIX + 1
T_WIN = T_PER_W + 8


def _body(src_hbm, out_hbm, src_v, buf0, buf1, sem0, sem1, src_sem):
    wid = lax.axis_index("s") * NC + lax.axis_index("c")
    zeros16_f = jnp.zeros((L,), jnp.float32)
    ones16_f = jnp.ones((L,), jnp.float32)
    zero16_i = jnp.zeros((L,), jnp.int32)
    lane = lax.iota(jnp.int32, L)
    t_loc = lane >> 2
    b_loc = lane & 3

    t_base = wid * T_PER_W
    t_lo = pl.multiple_of(jnp.maximum(t_base - 8, 0), 8)
    src_cp = pltpu.async_copy(src_hbm.at[pl.ds(t_lo, T_WIN)], src_v, src_sem)

    bufs = (buf0, buf1)
    sems = (sem0, sem1)

    def zero_buf(bf):
        @pl.loop(0, D_MODEL // L)
        def _zero(c):
            for tt in range(T_G):
                for bb in range(BATCH):
                    bf[tt, bb, pl.ds(c * L, L)] = zeros16_f

    def seg_cols_vals(t0):
        out = []
        for i in range(NSEG):
            t_idx = jnp.maximum(t0 - i + t_loc - t_lo, zero16_i)
            cols = plsc.load_gather(src_v, [t_idx, b_loc]) + (i * NTOKEN)
            vals = jnp.where(t0 + t_loc >= i, ones16_f, zeros16_f)
            out.append((cols, vals))
        return out

    def fill(bf, t0):
        for cols, vals in seg_cols_vals(t0):
            plsc.store_scatter(bf, [t_loc, b_loc, cols], vals)

    def clean(bf, t0_old):
        for cols, _ in seg_cols_vals(t0_old):
            plsc.store_scatter(bf, [t_loc, b_loc, cols], zeros16_f)

    def start(bf, t0, sem):
        return pltpu.async_copy(bf, out_hbm.at[pl.ds(t0, T_G)], sem)

    def drain(bf, t0, sem):
        pltpu.make_async_copy(bf, out_hbm.at[pl.ds(t0, T_G)], sem).wait()

    zero_buf(buf0)
    src_cp.wait()
    fill(buf0, t_base)
    start(buf0, t_base, sem0)
    zero_buf(buf1)
    fill(buf1, t_base + T_G)
    start(buf1, t_base + T_G, sem1)

    @pl.loop(1, GROUPS // 2)
    def _main(kk):
        for b in range(2):
            t0 = t_base + (kk * 2 + b) * T_G
            drain(bufs[b], t0 - 2 * T_G, sems[b])
            clean(bufs[b], t0 - 2 * T_G)
            fill(bufs[b], t0)
            start(bufs[b], t0, sems[b])

    t_last = t_base + (GROUPS - 2) * T_G
    drain(buf0, t_last, sem0)
    drain(buf1, t_last + T_G, sem1)


@jax.jit
def kernel(src):
    mesh = plsc.VectorSubcoreMesh(
        core_axis_name="c", subcore_axis_name="s", num_cores=NC, num_subcores=NS
    )
    k = pl.kernel(
        _body,
        out_type=jax.ShapeDtypeStruct((SEQ_LEN, BATCH, D_MODEL), jnp.float32),
        mesh=mesh,
        scratch_types=[
            pltpu.VMEM((T_WIN, BATCH), jnp.int32),
            pltpu.VMEM((T_G, BATCH, D_MODEL), jnp.float32),
            pltpu.VMEM((T_G, BATCH, D_MODEL), jnp.float32),
            pltpu.SemaphoreType.DMA,
            pltpu.SemaphoreType.DMA,
            pltpu.SemaphoreType.DMA,
        ],
        compiler_params=pltpu.CompilerParams(needs_layout_passes=False),
    )
    return k(src)


# speedup vs baseline: 3.0360x; 1.0013x over previous
import jax
import jax.numpy as jnp
from jax import lax
from jax.experimental import pallas as pl
from jax.experimental.pallas import tpu as pltpu
from jax.experimental.pallas import tpu_sc as plsc

NTOKEN = 256
MAX_PREFIX = 7
D_MODEL = 2048
SEQ_LEN = 2048
BATCH = 4

NC, NS, L = 2, 16, 16
NW = NC * NS
T_PER_W = SEQ_LEN // NW
T_G = L // BATCH
GROUPS = T_PER_W // T_G
NSEG = MAX_PREFIX + 1
T_WIN = T_PER_W + 8


def _body(src_hbm, out_hbm, src_v, buf0, buf1, sem0, sem1, src_sem):
    wid = lax.axis_index("s") * NC + lax.axis_index("c")
    zeros16_f = jnp.zeros((L,), jnp.float32)
    ones16_f = jnp.ones((L,), jnp.float32)
    zero16_i = jnp.zeros((L,), jnp.int32)
    lane = lax.iota(jnp.int32, L)
    t_loc = lane >> 2
    b_loc = lane & 3

    t_base = wid * T_PER_W
    t_lo = pl.multiple_of(jnp.maximum(t_base - 8, 0), 8)
    src_cp = pltpu.async_copy(src_hbm.at[pl.ds(t_lo, T_WIN)], src_v, src_sem)

    bufs = (buf0, buf1)
    sems = (sem0, sem1)

    def zero_buf(bf):
        @pl.loop(0, D_MODEL // L)
        def _zero(c):
            for tt in range(T_G):
                for bb in range(BATCH):
                    bf[tt, bb, pl.ds(c * L, L)] = zeros16_f

    def seg_cols_vals(t0):
        out = []
        for i in range(NSEG):
            t_idx = jnp.maximum(t0 - i + t_loc - t_lo, zero16_i)
            cols = plsc.load_gather(src_v, [t_idx, b_loc]) + (i * NTOKEN)
            vals = jnp.where(t0 + t_loc >= i, ones16_f, zeros16_f)
            out.append((cols, vals))
        return out

    def fill(bf, t0):
        for cols, vals in seg_cols_vals(t0):
            plsc.store_scatter(bf, [t_loc, b_loc, cols], vals)

    def clean(bf, t0_old):
        for cols, _ in seg_cols_vals(t0_old):
            plsc.store_scatter(bf, [t_loc, b_loc, cols], zeros16_f)

    def start(bf, t0, sem):
        return pltpu.async_copy(bf, out_hbm.at[pl.ds(t0, T_G)], sem)

    def drain(bf, t0, sem):
        pltpu.make_async_copy(bf, out_hbm.at[pl.ds(t0, T_G)], sem).wait()

    zero_buf(buf0)
    src_cp.wait()
    fill(buf0, t_base)
    start(buf0, t_base, sem0)
    zero_buf(buf1)
    fill(buf1, t_base + T_G)
    start(buf1, t_base + T_G, sem1)

    @pl.loop(1, GROUPS // 2)
    def _main(kk):
        for b in range(2):
            t0 = t_base + (kk * 2 + b) * T_G
            drain(bufs[b], t0 - 2 * T_G, sems[b])
            clean(bufs[b], t0 - 2 * T_G)
            fill(bufs[b], t0)
            start(bufs[b], t0, sems[b])

    t_last = t_base + (GROUPS - 2) * T_G
    drain(buf0, t_last, sem0)
    drain(buf1, t_last + T_G, sem1)


@jax.jit
def kernel(src):
    mesh = plsc.VectorSubcoreMesh(
        core_axis_name="c", subcore_axis_name="s", num_cores=NC, num_subcores=NS
    )
    k = pl.kernel(
        _body,
        out_type=jax.ShapeDtypeStruct((SEQ_LEN, BATCH, D_MODEL), jnp.float32),
        mesh=mesh,
        scratch_types=[
            pltpu.VMEM((T_WIN, BATCH), jnp.int32),
            pltpu.VMEM((T_G, BATCH, D_MODEL), jnp.float32),
            pltpu.VMEM((T_G, BATCH, D_MODEL), jnp.float32),
            pltpu.SemaphoreType.DMA,
            pltpu.SemaphoreType.DMA,
            pltpu.SemaphoreType.DMA,
        ],
        compiler_params=pltpu.CompilerParams(needs_layout_passes=False, skip_device_barrier=True),
    )
    return k(src)
